# baseline (speedup 1.0000x reference)
import functools

import jax
import jax.numpy as jnp
import numpy as np
from jax.experimental import pallas as pl

B, N, D, H = 16, 4096, 768, 192
K = 100
NC = 32
CW = 128
NB = 2048
BG = 4
INT_MIN = np.int32(-(2 ** 31))
INT_MAX = np.int32(2 ** 31 - 1)


def _scores_kernel(x_ref, wht_ref, bh_ref, ws_ref, bs_ref, o_ref):
    x2 = x_ref[0]
    pre = jax.lax.dot_general(
        x2, wht_ref[...], (((1,), (0,)), ((), ())),
        precision=jax.lax.Precision.HIGHEST,
        preferred_element_type=jnp.float32,
    ) + bh_ref[...]
    h = jax.nn.sigmoid(pre)
    h3 = h.reshape(NB // CW, CW, H)
    st = jnp.sum(h3 * ws_ref[...][None], axis=-1)
    o_ref[0] = st + bs_ref[0, 0]


def _select_kernel(s_ref, wm_ref, bm_ref, ext_ref, y_ref):
    s3 = s_ref[...]
    i32 = jax.lax.bitcast_convert_type(s3, jnp.int32)
    key = i32 ^ ((i32 >> 31) & np.int32(0x7FFFFFFF))
    k2 = jnp.concatenate([key, ~key], axis=0)
    r = 2 * BG

    cnt = jnp.sum((k2 >= 0).astype(jnp.int32), axis=(1, 2), keepdims=True)
    t = jnp.where(cnt >= K, jnp.int32(0), jnp.full((r, 1, 1), INT_MIN))
    for b in range(30, -1, -1):
        cand = t + np.int32(1 << b)
        cnt = jnp.sum((k2 >= cand).astype(jnp.int32), axis=(1, 2),
                      keepdims=True)
        t = jnp.where(cnt >= K, cand, t)

    strict = k2 > t
    c1 = jnp.sum(strict.astype(jnp.int32), axis=(1, 2), keepdims=False)
    c1 = c1.reshape(r, 1)
    t2 = t.reshape(r, 1)

    row_i = jax.lax.broadcasted_iota(jnp.int32, (CW, CW), 0)
    col_i = jax.lax.broadcasted_iota(jnp.int32, (CW, CW), 1)
    lt = (row_i < col_i).astype(jnp.float32)
    jsub = jax.lax.broadcasted_iota(jnp.int32, (r, CW, CW), 1).astype(
        jnp.float32)
    b02 = lambda v: jax.lax.broadcast_in_dim(v, (r, CW, CW), (0, 2))
    acc = jnp.zeros((r, CW), jnp.int32)
    running = jnp.zeros((r, 1), jnp.float32)
    for c in range(NC):
        sc_b = strict[:, c, :]
        sc_f = sc_b.astype(jnp.float32)
        excl = jax.lax.dot_general(
            sc_f, lt, (((1,), (0,)), ((), ())),
            preferred_element_type=jnp.float32)
        pos = excl + running
        oh = (b02(pos) == jsub) & b02(sc_b)
        contrib = jnp.sum(jnp.where(oh, b02(k2[:, c, :]), 0), axis=2)
        acc = acc + contrib
        running = running + jnp.sum(sc_f, axis=1, keepdims=True)

    slot = jax.lax.broadcasted_iota(jnp.int32, (r, CW), 1)
    compk = jnp.where(slot < c1, acc, t2)

    col3 = jax.lax.transpose(compk.reshape(r, 1, CW), (0, 2, 1))
    colb = jnp.broadcast_to(col3, (r, CW, CW))
    gt = b02(compk) > colb
    rank = jnp.sum(gt.astype(jnp.int32), axis=2)
    psub = jax.lax.broadcasted_iota(jnp.int32, (r, CW, CW), 1)
    cond = b02(rank) <= psub
    outk = jnp.min(jnp.where(cond, b02(compk), INT_MAX), axis=2)
    outk = outk[:, :K]

    topk = outk[:BG]
    botk = ~outk[BG:]
    inv = lambda kk: jax.lax.bitcast_convert_type(
        kk ^ ((kk >> 31) & np.int32(0x7FFFFFFF)), jnp.float32)
    ext = jnp.concatenate([inv(topk), inv(botk)], axis=1)
    ext_ref[0] = ext
    y_ref[0] = (jnp.sum(ext * wm_ref[...], axis=1, keepdims=True)
                + bm_ref[0, 0])


@jax.jit
def kernel(x, mask, Wh, bh, Ws, bs, Wm, bm):
    del mask
    wht = Wh.T
    bh2 = bh.reshape(1, H)
    ws2 = Ws.reshape(1, H)
    bs2 = bs.reshape(1, 1)
    scores = pl.pallas_call(
        _scores_kernel,
        grid=(B, N // NB),
        in_specs=[
            pl.BlockSpec((1, NB, D), lambda b, n: (b, n, 0)),
            pl.BlockSpec((D, H), lambda b, n: (0, 0)),
            pl.BlockSpec((1, H), lambda b, n: (0, 0)),
            pl.BlockSpec((1, H), lambda b, n: (0, 0)),
            pl.BlockSpec((1, 1), lambda b, n: (0, 0)),
        ],
        out_specs=pl.BlockSpec((1, NB // CW, CW), lambda b, n: (b, n, 0)),
        out_shape=jax.ShapeDtypeStruct((B, NC, CW), jnp.float32),
    )(x, wht, bh2, ws2, bs2)

    wm2 = Wm.reshape(1, 2 * K)
    bm2 = bm.reshape(1, 1)
    ext, y = pl.pallas_call(
        _select_kernel,
        grid=(B // BG,),
        in_specs=[
            pl.BlockSpec((BG, NC, CW), lambda g: (g, 0, 0)),
            pl.BlockSpec((1, 2 * K), lambda g: (0, 0)),
            pl.BlockSpec((1, 1), lambda g: (0, 0)),
        ],
        out_specs=(
            pl.BlockSpec((1, BG, 2 * K), lambda g: (g, 0, 0)),
            pl.BlockSpec((1, BG, 1), lambda g: (g, 0, 0)),
        ),
        out_shape=(
            jax.ShapeDtypeStruct((B // BG, BG, 2 * K), jnp.float32),
            jax.ShapeDtypeStruct((B // BG, BG, 1), jnp.float32),
        ),
    )(scores, wm2, bm2)
    return (y.reshape(B, 1), ext.reshape(B, 2 * K, 1))


# speedup vs baseline: 1.6286x; 1.6286x over previous
import functools

import jax
import jax.numpy as jnp
import numpy as np
from jax.experimental import pallas as pl

B, N, D, H = 16, 4096, 768, 192
K = 100
NC = 32
CW = 128
NB = 2048
BG = 4
INT_MIN = np.int32(-(2 ** 31))
INT_MAX = np.int32(2 ** 31 - 1)


def _scores_kernel(x_ref, wht_ref, bh_ref, ws_ref, bs_ref, o_ref):
    x2 = x_ref[0]
    pre = jax.lax.dot_general(
        x2, wht_ref[...], (((1,), (0,)), ((), ())),
        preferred_element_type=jnp.float32,
    ) + bh_ref[...]
    h = jax.nn.sigmoid(pre)
    h3 = h.reshape(NB // CW, CW, H)
    st = jnp.sum(h3 * ws_ref[...][None], axis=-1)
    o_ref[0] = st + bs_ref[0, 0]


def _select_kernel(s_ref, wm_ref, bm_ref, ext_ref, y_ref):
    s3 = s_ref[...]
    i32 = jax.lax.bitcast_convert_type(s3, jnp.int32)
    key = i32 ^ ((i32 >> 31) & np.int32(0x7FFFFFFF))
    k2 = jnp.concatenate([key, ~key], axis=0)
    r = 2 * BG

    cnt = jnp.sum((k2 >= 0).astype(jnp.int32), axis=(1, 2), keepdims=True)
    t = jnp.where(cnt >= K, jnp.int32(0), jnp.full((r, 1, 1), INT_MIN))
    for b in range(30, -1, -1):
        cand = t + np.int32(1 << b)
        cnt = jnp.sum((k2 >= cand).astype(jnp.int32), axis=(1, 2),
                      keepdims=True)
        t = jnp.where(cnt >= K, cand, t)

    strict = k2 > t
    c1 = jnp.sum(strict.astype(jnp.int32), axis=(1, 2), keepdims=False)
    c1 = c1.reshape(r, 1)
    t2 = t.reshape(r, 1)

    row_i = jax.lax.broadcasted_iota(jnp.int32, (CW, CW), 0)
    col_i = jax.lax.broadcasted_iota(jnp.int32, (CW, CW), 1)
    lt = (row_i < col_i).astype(jnp.float32)
    jsub = jax.lax.broadcasted_iota(jnp.int32, (r, CW, CW), 1).astype(
        jnp.float32)
    b02 = lambda v: jax.lax.broadcast_in_dim(v, (r, CW, CW), (0, 2))
    acc = jnp.zeros((r, CW), jnp.int32)
    running = jnp.zeros((r, 1), jnp.float32)
    for c in range(NC):
        sc_b = strict[:, c, :]
        sc_f = sc_b.astype(jnp.float32)
        excl = jax.lax.dot_general(
            sc_f, lt, (((1,), (0,)), ((), ())),
            preferred_element_type=jnp.float32)
        pos = excl + running
        oh = (b02(pos) == jsub) & b02(sc_b)
        contrib = jnp.sum(jnp.where(oh, b02(k2[:, c, :]), 0), axis=2)
        acc = acc + contrib
        running = running + jnp.sum(sc_f, axis=1, keepdims=True)

    slot = jax.lax.broadcasted_iota(jnp.int32, (r, CW), 1)
    compk = jnp.where(slot < c1, acc, t2)

    col3 = jax.lax.transpose(compk.reshape(r, 1, CW), (0, 2, 1))
    colb = jnp.broadcast_to(col3, (r, CW, CW))
    gt = b02(compk) > colb
    rank = jnp.sum(gt.astype(jnp.int32), axis=2)
    psub = jax.lax.broadcasted_iota(jnp.int32, (r, CW, CW), 1)
    cond = b02(rank) <= psub
    outk = jnp.min(jnp.where(cond, b02(compk), INT_MAX), axis=2)
    outk = outk[:, :K]

    topk = outk[:BG]
    botk = ~outk[BG:]
    inv = lambda kk: jax.lax.bitcast_convert_type(
        kk ^ ((kk >> 31) & np.int32(0x7FFFFFFF)), jnp.float32)
    ext = jnp.concatenate([inv(topk), inv(botk)], axis=1)
    ext_ref[0] = ext
    y_ref[0] = (jnp.sum(ext * wm_ref[...], axis=1, keepdims=True)
                + bm_ref[0, 0])


@jax.jit
def kernel(x, mask, Wh, bh, Ws, bs, Wm, bm):
    del mask
    wht = Wh.T
    bh2 = bh.reshape(1, H)
    ws2 = Ws.reshape(1, H)
    bs2 = bs.reshape(1, 1)
    scores = pl.pallas_call(
        _scores_kernel,
        grid=(B, N // NB),
        in_specs=[
            pl.BlockSpec((1, NB, D), lambda b, n: (b, n, 0)),
            pl.BlockSpec((D, H), lambda b, n: (0, 0)),
            pl.BlockSpec((1, H), lambda b, n: (0, 0)),
            pl.BlockSpec((1, H), lambda b, n: (0, 0)),
            pl.BlockSpec((1, 1), lambda b, n: (0, 0)),
        ],
        out_specs=pl.BlockSpec((1, NB // CW, CW), lambda b, n: (b, n, 0)),
        out_shape=jax.ShapeDtypeStruct((B, NC, CW), jnp.float32),
    )(x, wht, bh2, ws2, bs2)

    wm2 = Wm.reshape(1, 2 * K)
    bm2 = bm.reshape(1, 1)
    ext, y = pl.pallas_call(
        _select_kernel,
        grid=(B // BG,),
        in_specs=[
            pl.BlockSpec((BG, NC, CW), lambda g: (g, 0, 0)),
            pl.BlockSpec((1, 2 * K), lambda g: (0, 0)),
            pl.BlockSpec((1, 1), lambda g: (0, 0)),
        ],
        out_specs=(
            pl.BlockSpec((1, BG, 2 * K), lambda g: (g, 0, 0)),
            pl.BlockSpec((1, BG, 1), lambda g: (g, 0, 0)),
        ),
        out_shape=(
            jax.ShapeDtypeStruct((B // BG, BG, 2 * K), jnp.float32),
            jax.ShapeDtypeStruct((B // BG, BG, 1), jnp.float32),
        ),
    )(scores, wm2, bm2)
    return (y.reshape(B, 1), ext.reshape(B, 2 * K, 1))


# speedup vs baseline: 3.5742x; 2.1946x over previous
import jax
import jax.numpy as jnp
import numpy as np
from jax.experimental import pallas as pl
from jax.experimental.pallas import tpu as pltpu

B, N, D, H = 16, 4096, 768, 192
K = 100
NC = 32
CW = 128
G = 4


def _stage(v, p, keep_max):
    return jnp.where(keep_max, jnp.maximum(v, p), jnp.minimum(v, p))


def _sorted_top128(v):
    r = v.shape[0]
    lane = jax.lax.broadcasted_iota(jnp.int32, (r, NC, CW), 2)
    bitj = {j: (lane & np.int32(j)) != 0 for j in
            (1, 2, 4, 8, 16, 32, 64)}
    kclr = {k: (lane & np.int32(k)) == 0 for k in
            (2, 4, 8, 16, 32, 64)}

    def butterfly(vv, j):
        bj = bitj[j][:, : vv.shape[1]]
        return jnp.where(bj,
                         pltpu.roll(vv, j, 2),
                         pltpu.roll(vv, (CW - j) % CW, 2))

    d1 = jax.lax.broadcasted_iota(jnp.int32, (r, NC, CW), 1) >= NC // 2
    k = 2
    while k <= CW:
        j = k // 2
        while j >= 1:
            if k == CW:
                base = ~bitj[j] if j < CW else None
            else:
                base = kclr[k] ^ bitj[j]
            v = _stage(v, butterfly(v, j), base ^ d1)
            j //= 2
        k *= 2
    m = NC
    while m > 1:
        nl = m // 2
        a = v[:, :nl]
        b = v[:, nl:m]
        v = jnp.maximum(a, b)
        if nl > 1:
            da = jax.lax.broadcasted_iota(jnp.int32, a.shape, 1) >= nl // 2
        else:
            da = jnp.zeros(a.shape, jnp.bool_)
        j = CW // 2
        while j >= 1:
            keep_max = (~bitj[j][:, :nl]) ^ da
            v = _stage(v, butterfly(v, j), keep_max)
            j //= 2
        m //= 2
    return v[:, 0]


def _fused_kernel(x_ref, wht_ref, bh_ref, ws_ref, bs_ref, wm_ref, bm_ref,
                  ext_ref, y_ref, sc_ref):
    x2 = x_ref[0]
    pre = jax.lax.dot_general(
        x2, wht_ref[...], (((1,), (0,)), ((), ())),
        preferred_element_type=jnp.float32,
    ) + bh_ref[...]
    h = jax.nn.sigmoid(pre)
    h3 = h.reshape(NC, CW, H)
    st = jnp.sum(h3 * ws_ref[...][None], axis=-1) + bs_ref[0, 0]

    g = pl.program_id(0) % G
    sc_ref[g] = st

    @pl.when(g == G - 1)
    def _select():
        s3 = sc_ref[...]
        i32 = jax.lax.bitcast_convert_type(s3, jnp.int32)
        key = i32 ^ ((i32 >> 31) & np.int32(0x7FFFFFFF))
        v = jnp.concatenate([key, ~key], axis=0)
        outk = _sorted_top128(v)[:, :K]

        inv = lambda kk: jax.lax.bitcast_convert_type(
            kk ^ ((kk >> 31) & np.int32(0x7FFFFFFF)), jnp.float32)
        ext = jnp.concatenate([inv(outk[:G]), inv(~outk[G:])],
                              axis=1)
        ext_ref[0] = ext
        y_ref[0] = (jnp.sum(ext * wm_ref[...], axis=1, keepdims=True)
                    + bm_ref[0, 0])


@jax.jit
def kernel(x, mask, Wh, bh, Ws, bs, Wm, bm):
    del mask
    wht = Wh.T
    bh2 = bh.reshape(1, H)
    ws2 = Ws.reshape(1, H)
    bs2 = bs.reshape(1, 1)
    wm2 = Wm.reshape(1, 2 * K)
    bm2 = bm.reshape(1, 1)
    ext, y = pl.pallas_call(
        _fused_kernel,
        grid=(B,),
        in_specs=[
            pl.BlockSpec((1, N, D), lambda b: (b, 0, 0)),
            pl.BlockSpec((D, H), lambda b: (0, 0)),
            pl.BlockSpec((1, H), lambda b: (0, 0)),
            pl.BlockSpec((1, H), lambda b: (0, 0)),
            pl.BlockSpec((1, 1), lambda b: (0, 0)),
            pl.BlockSpec((1, 2 * K), lambda b: (0, 0)),
            pl.BlockSpec((1, 1), lambda b: (0, 0)),
        ],
        out_specs=(
            pl.BlockSpec((1, G, 2 * K), lambda b: (b // G, 0, 0)),
            pl.BlockSpec((1, G, 1), lambda b: (b // G, 0, 0)),
        ),
        out_shape=(
            jax.ShapeDtypeStruct((B // G, G, 2 * K), jnp.float32),
            jax.ShapeDtypeStruct((B // G, G, 1), jnp.float32),
        ),
        scratch_shapes=[pltpu.VMEM((G, NC, CW), jnp.float32)],
    )(x, wht, bh2, ws2, bs2, wm2, bm2)
    return (y.reshape(B, 1), ext.reshape(B, 2 * K, 1))


# speedup vs baseline: 3.8926x; 1.0891x over previous
import jax
import jax.numpy as jnp
import numpy as np
from jax.experimental import pallas as pl
from jax.experimental.pallas import tpu as pltpu

B, N, D, H = 16, 4096, 768, 192
K = 100
NC = 32
CW = 128
G = 8


def _stage(v, p, keep_max):
    return jnp.where(keep_max, jnp.maximum(v, p), jnp.minimum(v, p))


def _sorted_top128(v):
    r = v.shape[0]
    lane = jax.lax.broadcasted_iota(jnp.int32, (r, NC, CW), 2)
    bitj = {j: (lane & np.int32(j)) != 0 for j in
            (1, 2, 4, 8, 16, 32, 64)}
    kclr = {k: (lane & np.int32(k)) == 0 for k in
            (2, 4, 8, 16, 32, 64)}

    def butterfly(vv, j):
        bj = bitj[j][:, : vv.shape[1]]
        return jnp.where(bj,
                         pltpu.roll(vv, j, 2),
                         pltpu.roll(vv, (CW - j) % CW, 2))

    d1 = jax.lax.broadcasted_iota(jnp.int32, (r, NC, CW), 1) >= NC // 2
    k = 2
    while k <= CW:
        j = k // 2
        while j >= 1:
            if k == CW:
                base = ~bitj[j] if j < CW else None
            else:
                base = kclr[k] ^ bitj[j]
            v = _stage(v, butterfly(v, j), base ^ d1)
            j //= 2
        k *= 2
    m = NC
    while m > 1:
        nl = m // 2
        a = v[:, :nl]
        b = v[:, nl:m]
        v = jnp.maximum(a, b)
        if nl > 1:
            da = jax.lax.broadcasted_iota(jnp.int32, a.shape, 1) >= nl // 2
        else:
            da = jnp.zeros(a.shape, jnp.bool_)
        j = CW // 2
        while j >= 1:
            keep_max = (~bitj[j][:, :nl]) ^ da
            v = _stage(v, butterfly(v, j), keep_max)
            j //= 2
        m //= 2
    return v[:, 0]


def _fused_kernel(x_ref, wht_ref, bh_ref, ws_ref, bs_ref, wm_ref, bm_ref,
                  ext_ref, y_ref, sc_ref):
    x2 = x_ref[0]
    pre = jax.lax.dot_general(
        x2, wht_ref[...], (((1,), (0,)), ((), ())),
        preferred_element_type=jnp.float32,
    ) + bh_ref[...]
    h = jax.nn.sigmoid(pre)
    h3 = h.reshape(NC, CW, H)
    st = jnp.sum(h3 * ws_ref[...][None], axis=-1) + bs_ref[0, 0]

    g = pl.program_id(0) % G
    sc_ref[g] = st

    @pl.when(g == G - 1)
    def _select():
        s3 = sc_ref[...]
        i32 = jax.lax.bitcast_convert_type(s3, jnp.int32)
        key = i32 ^ ((i32 >> 31) & np.int32(0x7FFFFFFF))
        v = jnp.concatenate([key, ~key], axis=0)
        outk = _sorted_top128(v)[:, :K]

        inv = lambda kk: jax.lax.bitcast_convert_type(
            kk ^ ((kk >> 31) & np.int32(0x7FFFFFFF)), jnp.float32)
        ext = jnp.concatenate([inv(outk[:G]), inv(~outk[G:])],
                              axis=1)
        ext_ref[0] = ext
        y_ref[0] = (jnp.sum(ext * wm_ref[...], axis=1, keepdims=True)
                    + bm_ref[0, 0])


@jax.jit
def kernel(x, mask, Wh, bh, Ws, bs, Wm, bm):
    del mask
    wht = Wh.T
    bh2 = bh.reshape(1, H)
    ws2 = Ws.reshape(1, H)
    bs2 = bs.reshape(1, 1)
    wm2 = Wm.reshape(1, 2 * K)
    bm2 = bm.reshape(1, 1)
    ext, y = pl.pallas_call(
        _fused_kernel,
        grid=(B,),
        in_specs=[
            pl.BlockSpec((1, N, D), lambda b: (b, 0, 0)),
            pl.BlockSpec((D, H), lambda b: (0, 0)),
            pl.BlockSpec((1, H), lambda b: (0, 0)),
            pl.BlockSpec((1, H), lambda b: (0, 0)),
            pl.BlockSpec((1, 1), lambda b: (0, 0)),
            pl.BlockSpec((1, 2 * K), lambda b: (0, 0)),
            pl.BlockSpec((1, 1), lambda b: (0, 0)),
        ],
        out_specs=(
            pl.BlockSpec((1, G, 2 * K), lambda b: (b // G, 0, 0)),
            pl.BlockSpec((1, G, 1), lambda b: (b // G, 0, 0)),
        ),
        out_shape=(
            jax.ShapeDtypeStruct((B // G, G, 2 * K), jnp.float32),
            jax.ShapeDtypeStruct((B // G, G, 1), jnp.float32),
        ),
        scratch_shapes=[pltpu.VMEM((G, NC, CW), jnp.float32)],
    )(x, wht, bh2, ws2, bs2, wm2, bm2)
    return (y.reshape(B, 1), ext.reshape(B, 2 * K, 1))


# speedup vs baseline: 3.9773x; 1.0218x over previous
import jax
import jax.numpy as jnp
import numpy as np
from jax.experimental import pallas as pl
from jax.experimental.pallas import tpu as pltpu

B, N, D, H = 16, 4096, 768, 192
K = 100
NC = 32
CW = 128
G = 16


def _stage(v, p, keep_max):
    return jnp.where(keep_max, jnp.maximum(v, p), jnp.minimum(v, p))


def _sorted_top128(v):
    r = v.shape[0]
    lane = jax.lax.broadcasted_iota(jnp.int32, (r, NC, CW), 2)
    bitj = {j: (lane & np.int32(j)) != 0 for j in
            (1, 2, 4, 8, 16, 32, 64)}
    kclr = {k: (lane & np.int32(k)) == 0 for k in
            (2, 4, 8, 16, 32, 64)}

    def butterfly(vv, j):
        bj = bitj[j][:, : vv.shape[1]]
        return jnp.where(bj,
                         pltpu.roll(vv, j, 2),
                         pltpu.roll(vv, (CW - j) % CW, 2))

    d1 = jax.lax.broadcasted_iota(jnp.int32, (r, NC, CW), 1) >= NC // 2
    k = 2
    while k <= CW:
        j = k // 2
        while j >= 1:
            if k == CW:
                base = ~bitj[j] if j < CW else None
            else:
                base = kclr[k] ^ bitj[j]
            v = _stage(v, butterfly(v, j), base ^ d1)
            j //= 2
        k *= 2
    m = NC
    while m > 1:
        nl = m // 2
        a = v[:, :nl]
        b = v[:, nl:m]
        v = jnp.maximum(a, b)
        if nl > 1:
            da = jax.lax.broadcasted_iota(jnp.int32, a.shape, 1) >= nl // 2
        else:
            da = jnp.zeros(a.shape, jnp.bool_)
        j = CW // 2
        while j >= 1:
            keep_max = (~bitj[j][:, :nl]) ^ da
            v = _stage(v, butterfly(v, j), keep_max)
            j //= 2
        m //= 2
    return v[:, 0]


def _fused_kernel(x_ref, wht_ref, bh_ref, ws_ref, bs_ref, wm_ref, bm_ref,
                  ext_ref, y_ref, sc_ref):
    x2 = x_ref[0]
    pre = jax.lax.dot_general(
        x2, wht_ref[...], (((1,), (0,)), ((), ())),
        preferred_element_type=jnp.float32,
    ) + bh_ref[...]
    h = jax.nn.sigmoid(pre)
    h3 = h.reshape(NC, CW, H)
    st = jnp.sum(h3 * ws_ref[...][None], axis=-1) + bs_ref[0, 0]

    g = pl.program_id(0) % G
    sc_ref[g] = st

    @pl.when(g == G - 1)
    def _select():
        s3 = sc_ref[...]
        i32 = jax.lax.bitcast_convert_type(s3, jnp.int32)
        key = i32 ^ ((i32 >> 31) & np.int32(0x7FFFFFFF))
        v = jnp.concatenate([key, ~key], axis=0)
        outk = _sorted_top128(v)[:, :K]

        inv = lambda kk: jax.lax.bitcast_convert_type(
            kk ^ ((kk >> 31) & np.int32(0x7FFFFFFF)), jnp.float32)
        ext = jnp.concatenate([inv(outk[:G]), inv(~outk[G:])],
                              axis=1)
        ext_ref[0] = ext
        y_ref[0] = (jnp.sum(ext * wm_ref[...], axis=1, keepdims=True)
                    + bm_ref[0, 0])


@jax.jit
def kernel(x, mask, Wh, bh, Ws, bs, Wm, bm):
    del mask
    wht = Wh.T
    bh2 = bh.reshape(1, H)
    ws2 = Ws.reshape(1, H)
    bs2 = bs.reshape(1, 1)
    wm2 = Wm.reshape(1, 2 * K)
    bm2 = bm.reshape(1, 1)
    ext, y = pl.pallas_call(
        _fused_kernel,
        grid=(B,),
        in_specs=[
            pl.BlockSpec((1, N, D), lambda b: (b, 0, 0)),
            pl.BlockSpec((D, H), lambda b: (0, 0)),
            pl.BlockSpec((1, H), lambda b: (0, 0)),
            pl.BlockSpec((1, H), lambda b: (0, 0)),
            pl.BlockSpec((1, 1), lambda b: (0, 0)),
            pl.BlockSpec((1, 2 * K), lambda b: (0, 0)),
            pl.BlockSpec((1, 1), lambda b: (0, 0)),
        ],
        out_specs=(
            pl.BlockSpec((1, G, 2 * K), lambda b: (b // G, 0, 0)),
            pl.BlockSpec((1, G, 1), lambda b: (b // G, 0, 0)),
        ),
        out_shape=(
            jax.ShapeDtypeStruct((B // G, G, 2 * K), jnp.float32),
            jax.ShapeDtypeStruct((B // G, G, 1), jnp.float32),
        ),
        scratch_shapes=[pltpu.VMEM((G, NC, CW), jnp.float32)],
    )(x, wht, bh2, ws2, bs2, wm2, bm2)
    return (y.reshape(B, 1), ext.reshape(B, 2 * K, 1))
